# Initial kernel scaffold; baseline (speedup 1.0000x reference)
import jax
import jax.numpy as jnp
from jax.experimental import pallas as pl


def kernel(x, edges, emb, A1w, A1b, A2w, A2b, W1, b1, W2, b2):
    raise NotImplementedError("write your pallas kernel here")


# speedup vs baseline: 5.4152x; 5.4152x over previous
import functools

import jax
import jax.numpy as jnp
from jax import lax
from jax.experimental import pallas as pl
from jax.experimental.pallas import tpu as pltpu
from jax.experimental.pallas import tpu_sc as plsc

N = 10000
FEATS = 128
E = 320000
DEPTH = 10

NROWS = 10240
NPAD_ROWS = NROWS - N
R = NROWS // 16
BLK = 128
EP = 327680
NBLK = EP // BLK

_MESH = dict(core_axis_name="c", subcore_axis_name="s", num_cores=2,
             num_subcores=16)
_SC_PARAMS = pltpu.CompilerParams(use_tc_tiling_on_sc=False)


def _sc_degree(dst2d):
    mesh = plsc.VectorSubcoreMesh(**_MESH)

    @functools.partial(
        pl.kernel,
        out_type=jax.ShapeDtypeStruct((2, NROWS), jnp.float32),
        mesh=mesh,
        compiler_params=_SC_PARAMS,
        scratch_types=[
            pltpu.VMEM_SHARED((NROWS,), jnp.float32),
            pltpu.VMEM((R,), jnp.float32),
            pltpu.VMEM((BLK,), jnp.float32),
            pltpu.VMEM((BLK,), jnp.int32),
        ],
    )
    def k(dst_hbm, deg_hbm, deg_sh, zbuf, ones_v, idx_v):
        c = lax.axis_index("c")
        t = lax.axis_index("s")

        @pl.loop(0, R, step=16)
        def _(i):
            zbuf[pl.ds(i, 16)] = jnp.zeros((16,), jnp.float32)

        @pl.loop(0, BLK, step=16)
        def _(i):
            ones_v[pl.ds(i, 16)] = jnp.ones((16,), jnp.float32)

        pltpu.sync_copy(zbuf, deg_sh.at[pl.ds(t * R, R)])
        plsc.subcore_barrier()

        blocks_per_tile = NBLK // 32
        base = c * (NBLK // 2) + t * blocks_per_tile

        @pl.loop(0, blocks_per_tile)
        def _(i):
            pltpu.sync_copy(dst_hbm.at[base + i], idx_v)
            pltpu.sync_copy(ones_v, deg_sh.at[idx_v], add=True)

        plsc.subcore_barrier()
        pltpu.sync_copy(deg_sh.at[pl.ds(t * R, R)],
                        deg_hbm.at[c, pl.ds(t * R, R)])

    return k(dst2d)


def _sc_diffuse(xhat, s09, src2d, dst2d, feats, split_feats, pass_idx=0):
    mesh = plsc.VectorSubcoreMesh(**_MESH)
    if split_feats:
        io_t = jax.ShapeDtypeStruct((2, NROWS, feats), jnp.float32)
    else:
        io_t = jax.ShapeDtypeStruct((NROWS, feats), jnp.float32)
    nj = feats // 16
    blocks_per_tile = NBLK // 16

    @functools.partial(
        pl.kernel,
        out_type=io_t,
        mesh=mesh,
        compiler_params=_SC_PARAMS,
        scratch_types=[
            pltpu.VMEM_SHARED((NROWS, feats), jnp.float32),
            pltpu.VMEM_SHARED((NROWS, feats), jnp.float32),
            pltpu.VMEM((R, feats), jnp.float32),
            pltpu.VMEM((R, feats), jnp.float32),
            pltpu.VMEM((R,), jnp.float32),
            pltpu.VMEM((BLK, feats), jnp.float32),
            pltpu.VMEM((BLK,), jnp.int32),
            pltpu.VMEM((BLK,), jnp.int32),
        ],
    )
    def k(xh_hbm, s_hbm, src_hbm, dst_hbm, out_hbm,
          hh, acc, buf, h0s, s_v, rows, srcv, dstv):
        c = lax.axis_index("c")
        t = lax.axis_index("s")
        r0 = t * R

        if split_feats:
            pltpu.sync_copy(xh_hbm.at[2 * pass_idx + c, pl.ds(r0, R)], buf)
        else:
            pltpu.sync_copy(xh_hbm.at[pl.ds(r0, R)], buf)
        pltpu.sync_copy(s_hbm.at[pl.ds(r0, R)], s_v)

        @pl.loop(0, R)
        def _(r):
            for j in range(nj):
                h0s[r, pl.ds(j * 16, 16)] = buf[r, pl.ds(j * 16, 16)] * 0.1

        pltpu.sync_copy(buf, hh.at[pl.ds(r0, R)])
        plsc.subcore_barrier()

        @pl.loop(0, DEPTH)
        def _(it):
            @pl.loop(0, R)
            def _(r):
                for j in range(nj):
                    buf[r, pl.ds(j * 16, 16)] = jnp.zeros((16,), jnp.float32)

            pltpu.sync_copy(buf, acc.at[pl.ds(r0, R)])
            plsc.subcore_barrier()

            @pl.loop(0, blocks_per_tile)
            def _(i):
                blk = t * blocks_per_tile + i
                pltpu.sync_copy(src_hbm.at[blk], srcv)
                pltpu.sync_copy(dst_hbm.at[blk], dstv)
                pltpu.sync_copy(hh.at[srcv], rows)
                pltpu.sync_copy(rows, acc.at[dstv], add=True)

            plsc.subcore_barrier()
            pltpu.sync_copy(acc.at[pl.ds(r0, R)], buf)

            @pl.loop(0, R, step=16)
            def _(rg):
                sv = s_v[pl.ds(rg, 16)]
                for r16 in range(16):
                    sc = sv[r16]
                    r = rg + r16
                    for j in range(nj):
                        buf[r, pl.ds(j * 16, 16)] = (
                            buf[r, pl.ds(j * 16, 16)] * sc
                            + h0s[r, pl.ds(j * 16, 16)])

            pltpu.sync_copy(buf, hh.at[pl.ds(r0, R)])
            plsc.subcore_barrier()

        if split_feats:
            pltpu.sync_copy(hh.at[pl.ds(r0, R)],
                            out_hbm.at[c, pl.ds(r0, R)])
        else:
            half = NROWS // 2
            rh = half // 16
            pltpu.sync_copy(hh.at[pl.ds(c * half + t * rh, rh)],
                            out_hbm.at[pl.ds(c * half + t * rh, rh)])

    return k(xhat, s09, src2d, dst2d)


def _tc_prep(degp, xp):
    def body(degp_ref, x_ref, dis_ref, s09_ref, xhat_ref):
        deg = degp_ref[0] + degp_ref[1]
        dis = jnp.where(deg > 0, lax.rsqrt(jnp.maximum(deg, 1e-12)), 0.0)
        dis_ref[...] = dis
        s09_ref[...] = 0.9 * dis * dis
        xh = x_ref[...] * dis[:, None]
        for q in range(4):
            xhat_ref[q] = xh[:, 32 * q:32 * (q + 1)]

    return pl.pallas_call(
        body,
        out_shape=(
            jax.ShapeDtypeStruct((NROWS,), jnp.float32),
            jax.ShapeDtypeStruct((NROWS,), jnp.float32),
            jax.ShapeDtypeStruct((4, NROWS, 32), jnp.float32),
        ),
    )(degp, xp)


def _tc_mlp(hhat_a, hhat_b, dis, xp, emb, A1w, A1b, A2w, A2b, W1, b1, W2, b2):
    def body(hha_ref, hhb_ref, dis_ref, x_ref, emb_ref, A1w_ref, A1b_ref,
             A2w_ref, A2b_ref, W1_ref, b1_ref, W2_ref, b2_ref,
             z0_ref, zhat_ref):
        dis = dis_ref[...]
        x = x_ref[...]
        hh = jnp.concatenate(
            [hha_ref[0], hha_ref[1], hhb_ref[0], hhb_ref[1]], axis=1)
        inv = jnp.where(dis > 0, 1.0 / jnp.where(dis > 0, dis, 1.0), 0.0)
        h = hh * inv[:, None] + jnp.where(dis > 0, 0.0, 0.1)[:, None] * x

        A1w = A1w_ref[...]
        C = jnp.dot(emb_ref[...], A1w[2:],
                    preferred_element_type=jnp.float32) + A1b_ref[...]
        u = A1w[0]
        v = A1w[1]
        A2 = A2w_ref[...][:, 0]
        aacc = jnp.zeros_like(x)
        for j in range(12):
            aacc = aacc + jnp.maximum(
                h * u[j] + x * v[j] + C[:, j][None, :], 0.0) * A2[j]
        a = (aacc + A2b_ref[0]) * 0.5

        W1 = W1_ref[...]
        zr = jnp.maximum(
            jnp.dot(a, W1[:FEATS], preferred_element_type=jnp.float32)
            + jnp.dot(x, W1[FEATS:], preferred_element_type=jnp.float32)
            + b1_ref[...], 0.0)
        z0 = jnp.dot(zr, W2_ref[...],
                     preferred_element_type=jnp.float32) + b2_ref[...]
        z0_ref[...] = z0
        zhat_ref[...] = z0 * dis[:, None]

    return pl.pallas_call(
        body,
        out_shape=(
            jax.ShapeDtypeStruct((NROWS, 16), jnp.float32),
            jax.ShapeDtypeStruct((NROWS, 16), jnp.float32),
        ),
    )(hhat_a, hhat_b, dis, xp, emb, A1w, A1b, A2w, A2b, W1, b1, W2, b2)


def _tc_final(zhat2, dis, z0):
    def body(zh_ref, dis_ref, z0_ref, out_ref):
        dis = dis_ref[...]
        inv = jnp.where(dis > 0, 1.0 / jnp.where(dis > 0, dis, 1.0), 0.0)
        out_ref[...] = (zh_ref[...] * inv[:, None]
                        + jnp.where(dis > 0, 0.0, 0.1)[:, None] * z0_ref[...])

    return pl.pallas_call(
        body,
        out_shape=jax.ShapeDtypeStruct((NROWS, 16), jnp.float32),
    )(zhat2, dis, z0)


def kernel(x, edges, emb, A1w, A1b, A2w, A2b, W1, b1, W2, b2):
    src = edges[0]
    dst = edges[1]
    pade = EP - E
    ar = jnp.arange(pade, dtype=jnp.int32)
    src_p = jnp.concatenate([src, (ar * 997) % N])
    dst_p = jnp.concatenate([dst, N + (ar % NPAD_ROWS)])
    src2d = src_p.reshape(NBLK, BLK)
    dst2d = dst_p.reshape(NBLK, BLK)
    xp = jnp.pad(x, ((0, NROWS - N), (0, 0)))

    degp = _sc_degree(dst2d)
    dis, s09, xhat = _tc_prep(degp, xp)
    hhat_a = _sc_diffuse(xhat, s09, src2d, dst2d, feats=32,
                         split_feats=True, pass_idx=0)
    hhat_b = _sc_diffuse(xhat, s09, src2d, dst2d, feats=32,
                         split_feats=True, pass_idx=1)
    z0, zhat = _tc_mlp(hhat_a, hhat_b, dis, xp, emb, A1w, A1b, A2w, A2b,
                       W1, b1, W2, b2)
    zhat2 = _sc_diffuse(zhat, s09, src2d, dst2d, feats=16, split_feats=False)
    z = _tc_final(zhat2, dis, z0)
    return z[:N]
